# baseline (speedup 1.0000x reference)
import functools

import jax
import jax.numpy as jnp
from jax import lax
from jax.experimental import pallas as pl
from jax.experimental.pallas import tpu as pltpu
from jax.experimental.pallas import tpu_sc as plsc


def kernel(x, row_embed, col_embed):
    b, _, h, w = x.shape
    d = row_embed.shape[1]
    row_len = 2 * d * h * w
    nc, ns = 2, 16
    rows_per_w = b // (nc * ns)

    mesh = plsc.VectorSubcoreMesh(core_axis_name="c", subcore_axis_name="s")

    @functools.partial(
        pl.kernel,
        out_type=jax.ShapeDtypeStruct((b * row_len,), jnp.float32),
        mesh=mesh,
        scratch_types=[
            pltpu.VMEM((row_len,), jnp.float32),
            pltpu.SemaphoreType.DMA,
        ],
    )
    def _sc_fill(row_hbm, col_hbm, out_hbm, row_v, sem):
        segs = (
            (col_hbm, 0), (row_hbm, 0),
            (col_hbm, 1), (row_hbm, 0),
            (col_hbm, 0), (row_hbm, 1),
            (col_hbm, 1), (row_hbm, 1),
        )
        for s, (ref, r) in enumerate(segs):
            pltpu.sync_copy(ref.at[r], row_v.at[pl.ds(s * d, d)])
        wid = lax.axis_index("s") * nc + lax.axis_index("c")
        base = wid * rows_per_w
        cps = [
            pltpu.make_async_copy(
                row_v, out_hbm.at[pl.ds((base + r) * row_len, row_len)], sem
            )
            for r in range(rows_per_w)
        ]
        for c in cps:
            c.start()
        for c in cps:
            c.wait()

    out = _sc_fill(row_embed, col_embed)
    return out.reshape(b, h, w, 2 * d).transpose(0, 3, 1, 2)


# speedup vs baseline: 2.7394x; 2.7394x over previous
import jax
import jax.numpy as jnp
from jax.experimental import pallas as pl
from jax.experimental.pallas import tpu as pltpu

_BB = 16


def _pe_kernel(row_ref, col_ref, o_ref, scratch_ref, sem):
    col0 = col_ref[0:1, :]
    col1 = col_ref[1:2, :]
    row0 = row_ref[0:1, :]
    row1 = row_ref[1:2, :]
    row = jnp.concatenate(
        [col0, row0, col1, row0, col0, row1, col1, row1], axis=1
    )
    scratch_ref[...] = jnp.broadcast_to(row, scratch_ref.shape)

    b = o_ref.shape[0]
    copies = [
        pltpu.make_async_copy(
            scratch_ref, o_ref.at[pl.ds(t * _BB, _BB), :], sem
        )
        for t in range(b // _BB)
    ]
    for c in copies:
        c.start()
    for c in copies:
        c.wait()


def kernel(x, row_embed, col_embed):
    b, _, h, w = x.shape
    d = row_embed.shape[1]
    row_len = 2 * d * h * w
    out = pl.pallas_call(
        _pe_kernel,
        in_specs=[
            pl.BlockSpec(memory_space=pltpu.MemorySpace.VMEM),
            pl.BlockSpec(memory_space=pltpu.MemorySpace.VMEM),
        ],
        out_specs=pl.BlockSpec(memory_space=pl.ANY),
        out_shape=jax.ShapeDtypeStruct((b, row_len), x.dtype),
        scratch_shapes=[
            pltpu.VMEM((_BB, row_len), jnp.float32),
            pltpu.SemaphoreType.DMA,
        ],
        compiler_params=pltpu.CompilerParams(
            skip_device_barrier=True,
            disable_bounds_checks=True,
            disable_semaphore_checks=True,
        ),
    )(row_embed, col_embed)
    return out.reshape(b, h, w, 2 * d).transpose(0, 3, 1, 2)


# speedup vs baseline: 4.3463x; 1.5866x over previous
import jax
import jax.numpy as jnp
from jax.experimental import pallas as pl
from jax.experimental.pallas import tpu as pltpu

_BB = 8


def _pe_kernel(row_ref, col_ref, o_ref):
    col0 = col_ref[0]
    col1 = col_ref[1]
    row0 = row_ref[0]
    row1 = row_ref[1]
    m00 = jnp.concatenate([col0, row0], axis=0)
    m01 = jnp.concatenate([col1, row0], axis=0)
    m10 = jnp.concatenate([col0, row1], axis=0)
    m11 = jnp.concatenate([col1, row1], axis=0)
    c0 = jnp.stack([m00, m01], axis=1)
    c1 = jnp.stack([m10, m11], axis=1)
    cc = jnp.stack([c0, c1], axis=0)
    o_ref[...] = jnp.broadcast_to(cc[None], o_ref.shape)


def kernel(x, row_embed, col_embed):
    b, _, h, w = x.shape
    d = row_embed.shape[1]
    nt = 2 * d // 128
    col3 = col_embed.reshape(w, d // 128, 128)
    row3 = row_embed.reshape(h, d // 128, 128)
    out = pl.pallas_call(
        _pe_kernel,
        grid=(b // _BB,),
        in_specs=[
            pl.BlockSpec(row3.shape, lambda i: (0, 0, 0)),
            pl.BlockSpec(col3.shape, lambda i: (0, 0, 0)),
        ],
        out_specs=pl.BlockSpec((_BB, h, nt, w, 128), lambda i: (i, 0, 0, 0, 0)),
        out_shape=jax.ShapeDtypeStruct((b, h, nt, w, 128), x.dtype),
    )(row3, col3)
    return out.transpose(0, 2, 4, 1, 3).reshape(b, 2 * d, h, w)


# speedup vs baseline: 7.9524x; 1.8297x over previous
import jax
import jax.numpy as jnp
from jax.experimental import pallas as pl
from jax.experimental.pallas import tpu as pltpu

_BB = 16


def _pe_kernel(row_ref, col_ref, o_ref, scratch_ref, sem):
    col0 = col_ref[0]
    col1 = col_ref[1]
    row0 = row_ref[0]
    row1 = row_ref[1]
    m00 = jnp.concatenate([col0, row0], axis=0)
    m01 = jnp.concatenate([col1, row0], axis=0)
    m10 = jnp.concatenate([col0, row1], axis=0)
    m11 = jnp.concatenate([col1, row1], axis=0)
    c0 = jnp.stack([m00, m01], axis=1).reshape(64, 128)
    c1 = jnp.stack([m10, m11], axis=1).reshape(64, 128)
    cc = jnp.stack([c0, c1], axis=0)
    scratch_ref[...] = jnp.broadcast_to(cc[None], scratch_ref.shape)

    b = o_ref.shape[0]
    copies = [
        pltpu.make_async_copy(
            scratch_ref, o_ref.at[pl.ds(t * _BB, _BB)], sem
        )
        for t in range(b // _BB)
    ]
    for c in copies:
        c.start()
    for c in copies:
        c.wait()


def kernel(x, row_embed, col_embed):
    b, _, h, w = x.shape
    d = row_embed.shape[1]
    ntj = 2 * d * w // 128
    col3 = col_embed.reshape(w, d // 128, 128)
    row3 = row_embed.reshape(h, d // 128, 128)
    out = pl.pallas_call(
        _pe_kernel,
        in_specs=[
            pl.BlockSpec(memory_space=pltpu.MemorySpace.VMEM),
            pl.BlockSpec(memory_space=pltpu.MemorySpace.VMEM),
        ],
        out_specs=pl.BlockSpec(memory_space=pl.ANY),
        out_shape=jax.ShapeDtypeStruct((b, h, ntj, 128), x.dtype),
        scratch_shapes=[
            pltpu.VMEM((_BB, h, ntj, 128), jnp.float32),
            pltpu.SemaphoreType.DMA,
        ],
    )(row3, col3)
    out5 = out.reshape(b, h, ntj // w, w, 128)
    return out5.transpose(0, 2, 4, 1, 3).reshape(b, 2 * d, h, w)


# speedup vs baseline: 10.6267x; 1.3363x over previous
import jax
import jax.numpy as jnp
from jax.experimental import pallas as pl
from jax.experimental.pallas import tpu as pltpu

_BB = 16


def _chunks(arr2d, r, n):
    return jnp.concatenate(
        [
            jax.lax.slice(arr2d, (r, t * 128), (r + 1, (t + 1) * 128))
            for t in range(n)
        ],
        axis=0,
    )


def _pe_kernel(row_ref, col_ref, o_ref, scratch_ref, sem):
    rows = row_ref[...]
    cols = col_ref[...]
    n = cols.shape[1] // 128
    col0 = _chunks(cols, 0, n)
    col1 = _chunks(cols, 1, n)
    row0 = _chunks(rows, 0, n)
    row1 = _chunks(rows, 1, n)
    m00 = jnp.concatenate([col0, row0], axis=0)
    m01 = jnp.concatenate([col1, row0], axis=0)
    m10 = jnp.concatenate([col0, row1], axis=0)
    m11 = jnp.concatenate([col1, row1], axis=0)
    c0 = jnp.stack([m00, m01], axis=1).reshape(2 * m00.shape[0], 128)
    c1 = jnp.stack([m10, m11], axis=1).reshape(2 * m00.shape[0], 128)
    cc = jnp.stack([c0, c1], axis=0)
    scratch_ref[...] = jnp.broadcast_to(cc[None], scratch_ref.shape)

    b = o_ref.shape[0]
    copies = [
        pltpu.make_async_copy(
            scratch_ref, o_ref.at[pl.ds(t * _BB, _BB)], sem
        )
        for t in range(b // _BB)
    ]
    for c in copies:
        c.start()
    for c in copies:
        c.wait()


def kernel(x, row_embed, col_embed):
    b, _, h, w = x.shape
    d = row_embed.shape[1]
    ntj = 2 * d * w // 128
    out = pl.pallas_call(
        _pe_kernel,
        in_specs=[
            pl.BlockSpec(memory_space=pltpu.MemorySpace.VMEM),
            pl.BlockSpec(memory_space=pltpu.MemorySpace.VMEM),
        ],
        out_specs=pl.BlockSpec(memory_space=pl.ANY),
        out_shape=jax.ShapeDtypeStruct((b, h, ntj, 128), x.dtype),
        scratch_shapes=[
            pltpu.VMEM((_BB, h, ntj, 128), jnp.float32),
            pltpu.SemaphoreType.DMA,
        ],
    )(row_embed, col_embed)
    out5 = out.reshape(b, h, ntj // w, w, 128)
    return out5.transpose(0, 2, 4, 1, 3).reshape(b, 2 * d, h, w)
